# Initial kernel scaffold; baseline (speedup 1.0000x reference)
import jax
import jax.numpy as jnp
from jax.experimental import pallas as pl


def kernel(x, edge_index, edge_weight, batch, W1, b1, W2, b2, W3, b3, Wr, br):
    raise NotImplementedError("write your pallas kernel here")


# speedup vs baseline: 12.8705x; 12.8705x over previous
import functools

import jax
import jax.numpy as jnp
from jax import lax
from jax.experimental import pallas as pl
from jax.experimental.pallas import tpu as pltpu
from jax.experimental.pallas import tpu_sc as plsc

NC = 2
NS = 16
NW = NC * NS
LANES = 16
BLK = 128
G = 16
F32 = jnp.float32


def _mesh():
    return plsc.VectorSubcoreMesh(core_axis_name="c", subcore_axis_name="s")


def _deg_body(dst_h, w_h, out_h, dst_v, w_v, zbuf, acc_deg):
    c = lax.axis_index("c")
    s = lax.axis_index("s")
    wid = c * NS + s
    np_ = acc_deg.shape[0]
    rpt = np_ // NS
    pltpu.sync_copy(dst_h.at[wid], dst_v)
    pltpu.sync_copy(w_h.at[wid], w_v)

    z = jnp.zeros((LANES,), F32)

    def zero_body(i, _):
        zbuf[pl.ds(i * LANES, LANES)] = z
        return 0

    lax.fori_loop(0, rpt // LANES, zero_body, 0)
    pltpu.sync_copy(zbuf, acc_deg.at[pl.ds(s * rpt, rpt)])
    plsc.subcore_barrier()

    nblk = dst_v.shape[0]

    def blk_body(j, _):
        pltpu.sync_copy(w_v.at[j], acc_deg.at[dst_v.at[j]], add=True)
        return 0

    lax.fori_loop(0, nblk, blk_body, 0)
    plsc.subcore_barrier()
    pltpu.sync_copy(acc_deg.at[pl.ds(s * rpt, rpt)],
                    out_h.at[c, pl.ds(s * rpt, rpt)])


def _edge_body(src_h, dst_h, w_h, tab_h, out_h,
               src_v, dst_v, w_v, rows_v, acc, sem):
    c = lax.axis_index("c")
    s = lax.axis_index("s")
    wid = c * NS + s
    np_ = acc.shape[0]
    rpt = np_ // NS
    pltpu.sync_copy(src_h.at[wid], src_v)
    pltpu.sync_copy(dst_h.at[wid], dst_v)
    pltpu.sync_copy(w_h.at[wid], w_v)

    z = jnp.zeros((LANES,), F32)

    def zero_rows(i, _):
        for k in range(BLK // LANES):
            rows_v[i, pl.ds(k * LANES, LANES)] = z
        return 0

    lax.fori_loop(0, BLK, zero_rows, 0)
    for t in range(rpt // BLK):
        pltpu.sync_copy(rows_v, acc.at[pl.ds(s * rpt + t * BLK, BLK)])
    plsc.subcore_barrier()

    nblk = src_v.shape[0]

    def blk_body(j, _):
        pltpu.async_copy(tab_h.at[src_v.at[j]], rows_v, sem).wait()

        def grp(g, _):
            w16 = w_v[j, pl.ds(g * LANES, LANES)]
            for l in range(LANES):
                wsc = w16[l]
                e = g * LANES + l
                for k in range(BLK // LANES):
                    sl = pl.ds(k * LANES, LANES)
                    rows_v[e, sl] = rows_v[e, sl] * wsc
            return 0

        lax.fori_loop(0, BLK // LANES, grp, 0)
        pltpu.sync_copy(rows_v, acc.at[dst_v.at[j]], add=True)
        return 0

    lax.fori_loop(0, nblk, blk_body, 0)
    plsc.subcore_barrier()
    pltpu.sync_copy(acc.at[pl.ds(s * rpt, rpt)],
                    out_h.at[c, pl.ds(s * rpt, rpt)])


def _dinv_body(degp_ref, dinv_ref):
    deg = 1.0 + jnp.sum(degp_ref[...], axis=0, keepdims=True)
    dinv_ref[...] = jnp.where(deg > 0, lax.rsqrt(deg), 0.0)


def _m1_body(x_ref, dinv_ref, w_ref, out_ref):
    h = jnp.dot(x_ref[...], w_ref[...], preferred_element_type=F32,
                precision=lax.Precision.DEFAULT)
    out_ref[...] = dinv_ref[...] * h


def _mmid_body(pa_ref, pb_ref, hp_ref, dinv_ref, bias_ref, w_ref, out_ref):
    dinv = dinv_ref[...]
    o = dinv * (pa_ref[...] + pb_ref[...] + hp_ref[...]) + bias_ref[...]
    a = jnp.maximum(o, 0.0)
    h = jnp.dot(a, w_ref[...], preferred_element_type=F32,
                precision=lax.Precision.DEFAULT)
    out_ref[...] = dinv * h


def _pool_body(pa_ref, pb_ref, hp_ref, dinv_ref, bias_ref, batch_ref,
               sums_ref, cnts_ref):
    i = pl.program_id(0)
    o = dinv_ref[...] * (pa_ref[...] + pb_ref[...] + hp_ref[...]) + bias_ref[...]
    bt = batch_ref[...]
    gids = lax.broadcasted_iota(jnp.int32, (bt.shape[0], G), 1)
    onehot = (bt == gids).astype(F32)
    dn = (((0,), (0,)), ((), ()))
    ps = lax.dot_general(onehot, o, dn, precision=lax.Precision.HIGHEST)
    pc = lax.dot_general(onehot, jnp.ones_like(o), dn,
                         precision=lax.Precision.HIGHEST)

    @pl.when(i == 0)
    def _():
        sums_ref[...] = jnp.zeros_like(sums_ref)
        cnts_ref[...] = jnp.zeros_like(cnts_ref)

    sums_ref[...] += ps
    cnts_ref[...] += pc


def _head_body(sums_ref, cnts_ref, wr_ref, br_ref, out_ref):
    pooled = sums_ref[...] / jnp.maximum(cnts_ref[...], 1.0)
    o = jnp.dot(pooled, wr_ref[...], preferred_element_type=F32,
                precision=lax.Precision.DEFAULT) + br_ref[...]
    out_ref[...] = jnp.tanh(o)


def _row_spec(blk):
    return pl.BlockSpec(blk, lambda i: (i, 0))


def _const_spec(blk):
    return pl.BlockSpec(blk, lambda i: (0, 0))


def kernel(x, edge_index, edge_weight, batch, W1, b1, W2, b2, W3, b3, Wr, br):
    N, D = x.shape
    E = edge_index.shape[1]
    H = W1.shape[1]
    OUT = Wr.shape[1]

    NP = -(-N // (NS * BLK)) * (NS * BLK)
    CB = -(-E // (NW * BLK))
    EP = NW * CB * BLK

    x_p = jnp.zeros((NP, D), F32).at[:N].set(x)
    src = edge_index[0]
    dst = edge_index[1]
    pad_e = EP - E
    fill = jnp.arange(pad_e, dtype=jnp.int32) % N
    src_p = jnp.concatenate([src, fill]).reshape(NW, CB, BLK)
    dst_p = jnp.concatenate([dst, fill]).reshape(NW, CB, BLK)
    w_p = jnp.concatenate(
        [edge_weight, jnp.zeros((pad_e,), F32)]).reshape(NW, CB, BLK)
    batch_p = jnp.concatenate(
        [batch.astype(jnp.int32), jnp.full((NP - N,), G, jnp.int32)]
    ).reshape(NP, 1)

    deg_call = pl.kernel(
        _deg_body,
        out_type=jax.ShapeDtypeStruct((NC, NP), F32),
        mesh=_mesh(),
        scratch_types=[
            pltpu.VMEM((CB, BLK), jnp.int32),
            pltpu.VMEM((CB, BLK), F32),
            pltpu.VMEM((NP // NS,), F32),
            pltpu.VMEM_SHARED((NP,), F32),
        ],
    )
    degp = deg_call(dst_p, w_p)

    dinv_row = pl.pallas_call(
        _dinv_body,
        out_shape=jax.ShapeDtypeStruct((1, NP), F32),
    )(degp)
    dinv_col = dinv_row.reshape(NP, 1)

    edge_call = pl.kernel(
        _edge_body,
        out_type=jax.ShapeDtypeStruct((NC, NP, H), F32),
        mesh=_mesh(),
        scratch_types=[
            pltpu.VMEM((CB, BLK), jnp.int32),
            pltpu.VMEM((CB, BLK), jnp.int32),
            pltpu.VMEM((CB, BLK), F32),
            pltpu.VMEM((BLK, H), F32),
            pltpu.VMEM_SHARED((NP, H), F32),
            pltpu.SemaphoreType.DMA,
        ],
    )

    grid = (NP // BLK,)
    m1_call = pl.pallas_call(
        _m1_body,
        grid=grid,
        in_specs=[_row_spec((BLK, D)), _row_spec((BLK, 1)),
                  _const_spec((D, H))],
        out_specs=_row_spec((BLK, H)),
        out_shape=jax.ShapeDtypeStruct((NP, H), F32),
    )
    mmid_call = pl.pallas_call(
        _mmid_body,
        grid=grid,
        in_specs=[_row_spec((BLK, H)), _row_spec((BLK, H)),
                  _row_spec((BLK, H)), _row_spec((BLK, 1)),
                  _const_spec((1, H)), _const_spec((H, H))],
        out_specs=_row_spec((BLK, H)),
        out_shape=jax.ShapeDtypeStruct((NP, H), F32),
    )

    h1 = m1_call(x_p, dinv_col, W1)
    p1 = edge_call(src_p, dst_p, w_p, h1)
    h2 = mmid_call(p1[0], p1[1], h1, dinv_col, b1.reshape(1, H), W2)
    p2 = edge_call(src_p, dst_p, w_p, h2)
    h3 = mmid_call(p2[0], p2[1], h2, dinv_col, b2.reshape(1, H), W3)
    p3 = edge_call(src_p, dst_p, w_p, h3)

    sums, cnts = pl.pallas_call(
        _pool_body,
        grid=grid,
        in_specs=[_row_spec((BLK, H)), _row_spec((BLK, H)),
                  _row_spec((BLK, H)), _row_spec((BLK, 1)),
                  _const_spec((1, H)), _row_spec((BLK, 1))],
        out_specs=[_const_spec((G, H)), _const_spec((G, H))],
        out_shape=[jax.ShapeDtypeStruct((G, H), F32),
                   jax.ShapeDtypeStruct((G, H), F32)],
    )(p3[0], p3[1], h3, dinv_col, b3.reshape(1, H), batch_p)

    wr_pad = jnp.zeros((H, H), F32).at[:, :OUT].set(Wr)
    br_pad = jnp.zeros((1, H), F32).at[0, :OUT].set(br)
    out128 = pl.pallas_call(
        _head_body,
        out_shape=jax.ShapeDtypeStruct((G, H), F32),
    )(sums, cnts, wr_pad, br_pad)
    return out128[:, :OUT]


# speedup vs baseline: 17.3938x; 1.3515x over previous
import functools

import jax
import jax.numpy as jnp
from jax import lax
from jax.experimental import pallas as pl
from jax.experimental.pallas import tpu as pltpu
from jax.experimental.pallas import tpu_sc as plsc

NC = 2
NS = 16
NW = NC * NS
LANES = 16
BLK = 128
G = 16
F32 = jnp.float32


def _mesh():
    return plsc.VectorSubcoreMesh(core_axis_name="c", subcore_axis_name="s")


def _deg_body(dst_h, w_h, out_h, dst_v, w_v, zbuf, acc_deg):
    c = lax.axis_index("c")
    s = lax.axis_index("s")
    wid = c * NS + s
    np_ = acc_deg.shape[0]
    rpt = np_ // NS
    pltpu.sync_copy(dst_h.at[wid], dst_v)
    pltpu.sync_copy(w_h.at[wid], w_v)

    z = jnp.zeros((LANES,), F32)

    def zero_body(i, _):
        zbuf[pl.ds(i * LANES, LANES)] = z
        return 0

    lax.fori_loop(0, rpt // LANES, zero_body, 0)
    pltpu.sync_copy(zbuf, acc_deg.at[pl.ds(s * rpt, rpt)])
    plsc.subcore_barrier()

    nblk = dst_v.shape[0]

    def blk_body(j, _):
        pltpu.sync_copy(w_v.at[j], acc_deg.at[dst_v.at[j]], add=True)
        return 0

    lax.fori_loop(0, nblk, blk_body, 0)
    plsc.subcore_barrier()
    pltpu.sync_copy(acc_deg.at[pl.ds(s * rpt, rpt)],
                    out_h.at[c, pl.ds(s * rpt, rpt)])


CHUNK = 8
CPB = 2 * CHUNK


def _edge_body(src_h, dst_h, w_h, tab_h, out_h,
               rows0, rows1, src_c0, src_c1, dst_c0, dst_c1, w_c0, w_c1,
               acc, g0, g1, s0, s1, i0, i1):
    c = lax.axis_index("c")
    s = lax.axis_index("s")
    wid = c * NS + s
    np_ = acc.shape[0]
    rpt = np_ // NS
    cb = src_h.shape[1]
    ncp = cb // CPB

    rows = (rows0, rows1)
    srcc = (src_c0, src_c1)
    dstc = (dst_c0, dst_c1)
    wc = (w_c0, w_c1)
    gsem = (g0, g1)
    ssem = (s0, s1)
    isem = (i0, i1)

    z = jnp.zeros((LANES,), F32)

    def zero_rows(i, _):
        for k in range(BLK // LANES):
            rows0[i, pl.ds(k * LANES, LANES)] = z
        return 0

    lax.fori_loop(0, BLK, zero_rows, 0)
    for t in range(rpt // BLK):
        pltpu.sync_copy(rows0, acc.at[pl.ds(s * rpt + t * BLK, BLK)])
    plsc.subcore_barrier()

    def mult(buf, wref, k):
        def grp(g, _):
            w16 = wref[k, pl.ds(g * LANES, LANES)]
            for l in range(LANES):
                wsc = w16[l]
                e = g * LANES + l
                for q in range(BLK // LANES):
                    sl = pl.ds(q * LANES, LANES)
                    buf[e, sl] = buf[e, sl] * wsc
            return 0

        lax.fori_loop(0, BLK // LANES, grp, 0)

    def pf(cidx, st):
        pltpu.async_copy(src_h.at[wid, pl.ds(cidx * CHUNK, CHUNK)],
                         srcc[st], isem[st])
        pltpu.async_copy(dst_h.at[wid, pl.ds(cidx * CHUNK, CHUNK)],
                         dstc[st], isem[st])
        pltpu.async_copy(w_h.at[wid, pl.ds(cidx * CHUNK, CHUNK)],
                         wc[st], isem[st])

    def pf_wait(st):
        pltpu.make_async_copy(src_h.at[wid, pl.ds(0, CHUNK)],
                              srcc[st], isem[st]).wait()
        pltpu.make_async_copy(dst_h.at[wid, pl.ds(0, CHUNK)],
                              dstc[st], isem[st]).wait()
        pltpu.make_async_copy(w_h.at[wid, pl.ds(0, CHUNK)],
                              wc[st], isem[st]).wait()

    pltpu.sync_copy(src_h.at[wid, pl.ds(0, CHUNK)], src_c0)
    pltpu.sync_copy(dst_h.at[wid, pl.ds(0, CHUNK)], dst_c0)
    pltpu.sync_copy(w_h.at[wid, pl.ds(0, CHUNK)], w_c0)
    pltpu.async_copy(tab_h.at[src_c0.at[0]], rows0, g0)

    def cpair(cp, _):
        for jj in range(CPB):
            st = jj // CHUNK
            k = jj % CHUNK
            b = jj % 2
            pltpu.make_async_copy(tab_h.at[srcc[st].at[k]],
                                  rows[b], gsem[b]).wait()
            def drain():
                pltpu.make_async_copy(rows[1 - b], acc.at[dstc[st].at[k]],
                                      ssem[1 - b]).wait()
            if jj == 0:
                @pl.when(cp >= 1)
                def _():
                    drain()
            else:
                drain()
            if jj == 2:
                pf(2 * cp + 1, 1)
            if jj == 10:
                @pl.when(cp < ncp - 1)
                def _():
                    pf(2 * cp + 2, 0)
            if jj < CPB - 1:
                nst = (jj + 1) // CHUNK
                nk = (jj + 1) % CHUNK
                if jj == CHUNK - 1:
                    pf_wait(1)
                pltpu.async_copy(tab_h.at[srcc[nst].at[nk]],
                                 rows[1 - b], gsem[1 - b])
            else:
                @pl.when(cp < ncp - 1)
                def _():
                    pf_wait(0)
                    pltpu.async_copy(tab_h.at[src_c0.at[0]], rows[1 - b],
                                     gsem[1 - b])
            mult(rows[b], wc[st], k)
            pltpu.async_copy(rows[b], acc.at[dstc[st].at[k]], ssem[b],
                             add=True)
        return 0

    lax.fori_loop(0, ncp, cpair, 0)
    pltpu.make_async_copy(rows1, acc.at[dst_c1.at[CHUNK - 1]], s1).wait()
    plsc.subcore_barrier()
    pltpu.sync_copy(acc.at[pl.ds(s * rpt, rpt)],
                    out_h.at[c, pl.ds(s * rpt, rpt)])


def _dinv_body(degp_ref, dinv_ref):
    deg = 1.0 + jnp.sum(degp_ref[...], axis=0, keepdims=True)
    dinv_ref[...] = jnp.where(deg > 0, lax.rsqrt(deg), 0.0)


def _m1_body(x_ref, dinv_ref, w_ref, out_ref):
    h = jnp.dot(x_ref[...], w_ref[...], preferred_element_type=F32,
                precision=lax.Precision.DEFAULT)
    out_ref[...] = dinv_ref[...] * h


def _mmid_body(pa_ref, pb_ref, hp_ref, dinv_ref, bias_ref, w_ref, out_ref):
    dinv = dinv_ref[...]
    o = dinv * (pa_ref[...] + pb_ref[...] + hp_ref[...]) + bias_ref[...]
    a = jnp.maximum(o, 0.0)
    h = jnp.dot(a, w_ref[...], preferred_element_type=F32,
                precision=lax.Precision.DEFAULT)
    out_ref[...] = dinv * h


def _pool_body(pa_ref, pb_ref, hp_ref, dinv_ref, bias_ref, batch_ref,
               sums_ref, cnts_ref):
    i = pl.program_id(0)
    o = dinv_ref[...] * (pa_ref[...] + pb_ref[...] + hp_ref[...]) + bias_ref[...]
    bt = batch_ref[...]
    gids = lax.broadcasted_iota(jnp.int32, (bt.shape[0], G), 1)
    onehot = (bt == gids).astype(F32)
    dn = (((0,), (0,)), ((), ()))
    ps = lax.dot_general(onehot, o, dn, precision=lax.Precision.HIGHEST)
    pc = lax.dot_general(onehot, jnp.ones_like(o), dn,
                         precision=lax.Precision.HIGHEST)

    @pl.when(i == 0)
    def _():
        sums_ref[...] = jnp.zeros_like(sums_ref)
        cnts_ref[...] = jnp.zeros_like(cnts_ref)

    sums_ref[...] += ps
    cnts_ref[...] += pc


def _head_body(sums_ref, cnts_ref, wr_ref, br_ref, out_ref):
    pooled = sums_ref[...] / jnp.maximum(cnts_ref[...], 1.0)
    o = jnp.dot(pooled, wr_ref[...], preferred_element_type=F32,
                precision=lax.Precision.DEFAULT) + br_ref[...]
    out_ref[...] = jnp.tanh(o)


def _row_spec(blk):
    return pl.BlockSpec(blk, lambda i: (i, 0))


def _const_spec(blk):
    return pl.BlockSpec(blk, lambda i: (0, 0))


def kernel(x, edge_index, edge_weight, batch, W1, b1, W2, b2, W3, b3, Wr, br):
    N, D = x.shape
    E = edge_index.shape[1]
    H = W1.shape[1]
    OUT = Wr.shape[1]

    NP = -(-N // (NS * BLK)) * (NS * BLK)
    CB = CPB * -(-E // (NW * BLK * CPB))
    EP = NW * CB * BLK

    x_p = jnp.zeros((NP, D), F32).at[:N].set(x)
    src = edge_index[0]
    dst = edge_index[1]
    pad_e = EP - E
    fill = jnp.arange(pad_e, dtype=jnp.int32) % N
    src_p = jnp.concatenate([src, fill]).reshape(NW, CB, BLK)
    dst_p = jnp.concatenate([dst, fill]).reshape(NW, CB, BLK)
    w_p = jnp.concatenate(
        [edge_weight, jnp.zeros((pad_e,), F32)]).reshape(NW, CB, BLK)
    batch_p = jnp.concatenate(
        [batch.astype(jnp.int32), jnp.full((NP - N,), G, jnp.int32)]
    ).reshape(NP, 1)

    deg_call = pl.kernel(
        _deg_body,
        out_type=jax.ShapeDtypeStruct((NC, NP), F32),
        mesh=_mesh(),
        scratch_types=[
            pltpu.VMEM((CB, BLK), jnp.int32),
            pltpu.VMEM((CB, BLK), F32),
            pltpu.VMEM((NP // NS,), F32),
            pltpu.VMEM_SHARED((NP,), F32),
        ],
    )
    degp = deg_call(dst_p, w_p)

    dinv_row = pl.pallas_call(
        _dinv_body,
        out_shape=jax.ShapeDtypeStruct((1, NP), F32),
    )(degp)
    dinv_col = dinv_row.reshape(NP, 1)

    edge_call = pl.kernel(
        _edge_body,
        out_type=jax.ShapeDtypeStruct((NC, NP, H), F32),
        mesh=_mesh(),
        scratch_types=[
            pltpu.VMEM((BLK, H), F32),
            pltpu.VMEM((BLK, H), F32),
            pltpu.VMEM((CHUNK, BLK), jnp.int32),
            pltpu.VMEM((CHUNK, BLK), jnp.int32),
            pltpu.VMEM((CHUNK, BLK), jnp.int32),
            pltpu.VMEM((CHUNK, BLK), jnp.int32),
            pltpu.VMEM((CHUNK, BLK), F32),
            pltpu.VMEM((CHUNK, BLK), F32),
            pltpu.VMEM_SHARED((NP, H), F32),
            pltpu.SemaphoreType.DMA,
            pltpu.SemaphoreType.DMA,
            pltpu.SemaphoreType.DMA,
            pltpu.SemaphoreType.DMA,
            pltpu.SemaphoreType.DMA,
            pltpu.SemaphoreType.DMA,
        ],
    )

    grid = (NP // BLK,)
    m1_call = pl.pallas_call(
        _m1_body,
        grid=grid,
        in_specs=[_row_spec((BLK, D)), _row_spec((BLK, 1)),
                  _const_spec((D, H))],
        out_specs=_row_spec((BLK, H)),
        out_shape=jax.ShapeDtypeStruct((NP, H), F32),
    )
    mmid_call = pl.pallas_call(
        _mmid_body,
        grid=grid,
        in_specs=[_row_spec((BLK, H)), _row_spec((BLK, H)),
                  _row_spec((BLK, H)), _row_spec((BLK, 1)),
                  _const_spec((1, H)), _const_spec((H, H))],
        out_specs=_row_spec((BLK, H)),
        out_shape=jax.ShapeDtypeStruct((NP, H), F32),
    )

    h1 = m1_call(x_p, dinv_col, W1)
    p1 = edge_call(src_p, dst_p, w_p, h1)
    h2 = mmid_call(p1[0], p1[1], h1, dinv_col, b1.reshape(1, H), W2)
    p2 = edge_call(src_p, dst_p, w_p, h2)
    h3 = mmid_call(p2[0], p2[1], h2, dinv_col, b2.reshape(1, H), W3)
    p3 = edge_call(src_p, dst_p, w_p, h3)

    sums, cnts = pl.pallas_call(
        _pool_body,
        grid=grid,
        in_specs=[_row_spec((BLK, H)), _row_spec((BLK, H)),
                  _row_spec((BLK, H)), _row_spec((BLK, 1)),
                  _const_spec((1, H)), _row_spec((BLK, 1))],
        out_specs=[_const_spec((G, H)), _const_spec((G, H))],
        out_shape=[jax.ShapeDtypeStruct((G, H), F32),
                   jax.ShapeDtypeStruct((G, H), F32)],
    )(p3[0], p3[1], h3, dinv_col, b3.reshape(1, H), batch_p)

    wr_pad = jnp.zeros((H, H), F32).at[:, :OUT].set(Wr)
    br_pad = jnp.zeros((1, H), F32).at[0, :OUT].set(br)
    out128 = pl.pallas_call(
        _head_body,
        out_shape=jax.ShapeDtypeStruct((G, H), F32),
    )(sums, cnts, wr_pad, br_pad)
    return out128[:, :OUT]


# speedup vs baseline: 17.5839x; 1.0109x over previous
import functools

import jax
import jax.numpy as jnp
from jax import lax
from jax.experimental import pallas as pl
from jax.experimental.pallas import tpu as pltpu
from jax.experimental.pallas import tpu_sc as plsc

NC = 2
NS = 16
NW = NC * NS
LANES = 16
BLK = 128
G = 16
F32 = jnp.float32


def _mesh():
    return plsc.VectorSubcoreMesh(core_axis_name="c", subcore_axis_name="s")


def _deg_body(dst_h, w_h, out_h, dst_v, w_v, zbuf, acc_deg):
    c = lax.axis_index("c")
    s = lax.axis_index("s")
    wid = c * NS + s
    np_ = acc_deg.shape[0]
    rpt = np_ // NS
    pltpu.sync_copy(dst_h.at[wid], dst_v)
    pltpu.sync_copy(w_h.at[wid], w_v)

    z = jnp.zeros((LANES,), F32)

    def zero_body(i, _):
        zbuf[pl.ds(i * LANES, LANES)] = z
        return 0

    lax.fori_loop(0, rpt // LANES, zero_body, 0)
    pltpu.sync_copy(zbuf, acc_deg.at[pl.ds(s * rpt, rpt)])
    plsc.subcore_barrier()

    nblk = dst_v.shape[0]

    def blk_body(j, _):
        pltpu.sync_copy(w_v.at[j], acc_deg.at[dst_v.at[j]], add=True)
        return 0

    lax.fori_loop(0, nblk, blk_body, 0)
    plsc.subcore_barrier()
    pltpu.sync_copy(acc_deg.at[pl.ds(s * rpt, rpt)],
                    out_h.at[c, pl.ds(s * rpt, rpt)])


CHUNK = 8
CPB = 2 * CHUNK


def _edge_body(src_h, dst_h, w_h, tab_h, out_h,
               rows0, rows1, src_c0, src_c1, dst_c0, dst_c1, w_c0, w_c1,
               acc, g0, g1, h0, h1, s0, s1, i0, i1):
    c = lax.axis_index("c")
    s = lax.axis_index("s")
    wid = c * NS + s
    np_ = acc.shape[0]
    rpt = np_ // NS
    cb = src_h.shape[1]
    ncp = cb // CPB

    rows = (rows0, rows1)
    srcc = (src_c0, src_c1)
    dstc = (dst_c0, dst_c1)
    wc = (w_c0, w_c1)
    gsem = (g0, g1)
    hsem = (h0, h1)
    ssem = (s0, s1)
    isem = (i0, i1)

    z = jnp.zeros((LANES,), F32)

    def zero_rows(i, _):
        for k in range(BLK // LANES):
            rows0[i, pl.ds(k * LANES, LANES)] = z
        return 0

    lax.fori_loop(0, BLK, zero_rows, 0)
    for t in range(rpt // BLK):
        pltpu.sync_copy(rows0, acc.at[pl.ds(s * rpt + t * BLK, BLK)])
    plsc.subcore_barrier()

    def mult(buf, wref, k):
        def grp(g, _):
            w16 = wref[k, pl.ds(g * LANES, LANES)]
            for l in range(LANES):
                wsc = w16[l]
                e = g * LANES + l
                for q in range(BLK // LANES):
                    sl = pl.ds(q * LANES, LANES)
                    buf[e, sl] = buf[e, sl] * wsc
            return 0

        lax.fori_loop(0, BLK // LANES, grp, 0)

    def pf(cidx, st):
        pltpu.async_copy(src_h.at[wid, pl.ds(cidx * CHUNK, CHUNK)],
                         srcc[st], isem[st])
        pltpu.async_copy(dst_h.at[wid, pl.ds(cidx * CHUNK, CHUNK)],
                         dstc[st], isem[st])
        pltpu.async_copy(w_h.at[wid, pl.ds(cidx * CHUNK, CHUNK)],
                         wc[st], isem[st])

    def pf_wait(st):
        pltpu.make_async_copy(src_h.at[wid, pl.ds(0, CHUNK)],
                              srcc[st], isem[st]).wait()
        pltpu.make_async_copy(dst_h.at[wid, pl.ds(0, CHUNK)],
                              dstc[st], isem[st]).wait()
        pltpu.make_async_copy(w_h.at[wid, pl.ds(0, CHUNK)],
                              wc[st], isem[st]).wait()

    pltpu.sync_copy(src_h.at[wid, pl.ds(0, CHUNK)], src_c0)
    pltpu.sync_copy(dst_h.at[wid, pl.ds(0, CHUNK)], dst_c0)
    pltpu.sync_copy(w_h.at[wid, pl.ds(0, CHUNK)], w_c0)
    HB = BLK // 2

    def g_issue(idxrow, buf, bi):
        pltpu.async_copy(tab_h.at[idxrow.at[pl.ds(0, HB)]],
                         buf.at[pl.ds(0, HB)], gsem[bi])
        pltpu.async_copy(tab_h.at[idxrow.at[pl.ds(HB, HB)]],
                         buf.at[pl.ds(HB, HB)], hsem[bi])

    def g_wait(idxrow, buf, bi):
        pltpu.make_async_copy(tab_h.at[idxrow.at[pl.ds(0, HB)]],
                              buf.at[pl.ds(0, HB)], gsem[bi]).wait()
        pltpu.make_async_copy(tab_h.at[idxrow.at[pl.ds(HB, HB)]],
                              buf.at[pl.ds(HB, HB)], hsem[bi]).wait()

    g_issue(src_c0.at[0], rows0, 0)

    def cpair(cp, _):
        for jj in range(CPB):
            st = jj // CHUNK
            k = jj % CHUNK
            b = jj % 2
            g_wait(srcc[st].at[k], rows[b], b)
            def drain():
                pltpu.make_async_copy(rows[1 - b], acc.at[dstc[st].at[k]],
                                      ssem[1 - b]).wait()
            if jj == 0:
                @pl.when(cp >= 1)
                def _():
                    drain()
            else:
                drain()
            if jj == 2:
                pf(2 * cp + 1, 1)
            if jj == 10:
                @pl.when(cp < ncp - 1)
                def _():
                    pf(2 * cp + 2, 0)
            if jj < CPB - 1:
                nst = (jj + 1) // CHUNK
                nk = (jj + 1) % CHUNK
                if jj == CHUNK - 1:
                    pf_wait(1)
                g_issue(srcc[nst].at[nk], rows[1 - b], 1 - b)
            else:
                @pl.when(cp < ncp - 1)
                def _():
                    pf_wait(0)
                    g_issue(src_c0.at[0], rows[1 - b], 1 - b)
            mult(rows[b], wc[st], k)
            pltpu.async_copy(rows[b], acc.at[dstc[st].at[k]], ssem[b],
                             add=True)
        return 0

    lax.fori_loop(0, ncp, cpair, 0)
    pltpu.make_async_copy(rows1, acc.at[dst_c1.at[CHUNK - 1]], s1).wait()
    plsc.subcore_barrier()
    pltpu.sync_copy(acc.at[pl.ds(s * rpt, rpt)],
                    out_h.at[c, pl.ds(s * rpt, rpt)])


def _dinv_body(degp_ref, dinv_ref):
    deg = 1.0 + jnp.sum(degp_ref[...], axis=0, keepdims=True)
    dinv_ref[...] = jnp.where(deg > 0, lax.rsqrt(deg), 0.0)


def _m1_body(x_ref, dinv_ref, w_ref, out_ref):
    h = jnp.dot(x_ref[...], w_ref[...], preferred_element_type=F32,
                precision=lax.Precision.DEFAULT)
    out_ref[...] = dinv_ref[...] * h


def _mmid_body(pa_ref, pb_ref, hp_ref, dinv_ref, bias_ref, w_ref, out_ref):
    dinv = dinv_ref[...]
    o = dinv * (pa_ref[...] + pb_ref[...] + hp_ref[...]) + bias_ref[...]
    a = jnp.maximum(o, 0.0)
    h = jnp.dot(a, w_ref[...], preferred_element_type=F32,
                precision=lax.Precision.DEFAULT)
    out_ref[...] = dinv * h


def _pool_body(pa_ref, pb_ref, hp_ref, dinv_ref, bias_ref, batch_ref,
               sums_ref, cnts_ref):
    i = pl.program_id(0)
    o = dinv_ref[...] * (pa_ref[...] + pb_ref[...] + hp_ref[...]) + bias_ref[...]
    bt = batch_ref[...]
    gids = lax.broadcasted_iota(jnp.int32, (bt.shape[0], G), 1)
    onehot = (bt == gids).astype(F32)
    dn = (((0,), (0,)), ((), ()))
    ps = lax.dot_general(onehot, o, dn, precision=lax.Precision.HIGHEST)
    pc = lax.dot_general(onehot, jnp.ones_like(o), dn,
                         precision=lax.Precision.HIGHEST)

    @pl.when(i == 0)
    def _():
        sums_ref[...] = jnp.zeros_like(sums_ref)
        cnts_ref[...] = jnp.zeros_like(cnts_ref)

    sums_ref[...] += ps
    cnts_ref[...] += pc


def _head_body(sums_ref, cnts_ref, wr_ref, br_ref, out_ref):
    pooled = sums_ref[...] / jnp.maximum(cnts_ref[...], 1.0)
    o = jnp.dot(pooled, wr_ref[...], preferred_element_type=F32,
                precision=lax.Precision.DEFAULT) + br_ref[...]
    out_ref[...] = jnp.tanh(o)


def _row_spec(blk):
    return pl.BlockSpec(blk, lambda i: (i, 0))


def _const_spec(blk):
    return pl.BlockSpec(blk, lambda i: (0, 0))


def kernel(x, edge_index, edge_weight, batch, W1, b1, W2, b2, W3, b3, Wr, br):
    N, D = x.shape
    E = edge_index.shape[1]
    H = W1.shape[1]
    OUT = Wr.shape[1]

    NP = -(-N // (NS * BLK)) * (NS * BLK)
    CB = CPB * -(-E // (NW * BLK * CPB))
    EP = NW * CB * BLK

    x_p = jnp.zeros((NP, D), F32).at[:N].set(x)
    src = edge_index[0]
    dst = edge_index[1]
    pad_e = EP - E
    fill = jnp.arange(pad_e, dtype=jnp.int32) % N
    src_p = jnp.concatenate([src, fill]).reshape(NW, CB, BLK)
    dst_p = jnp.concatenate([dst, fill]).reshape(NW, CB, BLK)
    w_p = jnp.concatenate(
        [edge_weight, jnp.zeros((pad_e,), F32)]).reshape(NW, CB, BLK)
    batch_p = jnp.concatenate(
        [batch.astype(jnp.int32), jnp.full((NP - N,), G, jnp.int32)]
    ).reshape(NP, 1)

    deg_call = pl.kernel(
        _deg_body,
        out_type=jax.ShapeDtypeStruct((NC, NP), F32),
        mesh=_mesh(),
        scratch_types=[
            pltpu.VMEM((CB, BLK), jnp.int32),
            pltpu.VMEM((CB, BLK), F32),
            pltpu.VMEM((NP // NS,), F32),
            pltpu.VMEM_SHARED((NP,), F32),
        ],
    )
    degp = deg_call(dst_p, w_p)

    dinv_row = pl.pallas_call(
        _dinv_body,
        out_shape=jax.ShapeDtypeStruct((1, NP), F32),
    )(degp)
    dinv_col = dinv_row.reshape(NP, 1)

    edge_call = pl.kernel(
        _edge_body,
        out_type=jax.ShapeDtypeStruct((NC, NP, H), F32),
        mesh=_mesh(),
        scratch_types=[
            pltpu.VMEM((BLK, H), F32),
            pltpu.VMEM((BLK, H), F32),
            pltpu.VMEM((CHUNK, BLK), jnp.int32),
            pltpu.VMEM((CHUNK, BLK), jnp.int32),
            pltpu.VMEM((CHUNK, BLK), jnp.int32),
            pltpu.VMEM((CHUNK, BLK), jnp.int32),
            pltpu.VMEM((CHUNK, BLK), F32),
            pltpu.VMEM((CHUNK, BLK), F32),
            pltpu.VMEM_SHARED((NP, H), F32),
            pltpu.SemaphoreType.DMA,
            pltpu.SemaphoreType.DMA,
            pltpu.SemaphoreType.DMA,
            pltpu.SemaphoreType.DMA,
            pltpu.SemaphoreType.DMA,
            pltpu.SemaphoreType.DMA,
            pltpu.SemaphoreType.DMA,
            pltpu.SemaphoreType.DMA,
        ],
    )

    grid = (NP // BLK,)
    m1_call = pl.pallas_call(
        _m1_body,
        grid=grid,
        in_specs=[_row_spec((BLK, D)), _row_spec((BLK, 1)),
                  _const_spec((D, H))],
        out_specs=_row_spec((BLK, H)),
        out_shape=jax.ShapeDtypeStruct((NP, H), F32),
    )
    mmid_call = pl.pallas_call(
        _mmid_body,
        grid=grid,
        in_specs=[_row_spec((BLK, H)), _row_spec((BLK, H)),
                  _row_spec((BLK, H)), _row_spec((BLK, 1)),
                  _const_spec((1, H)), _const_spec((H, H))],
        out_specs=_row_spec((BLK, H)),
        out_shape=jax.ShapeDtypeStruct((NP, H), F32),
    )

    h1 = m1_call(x_p, dinv_col, W1)
    p1 = edge_call(src_p, dst_p, w_p, h1)
    h2 = mmid_call(p1[0], p1[1], h1, dinv_col, b1.reshape(1, H), W2)
    p2 = edge_call(src_p, dst_p, w_p, h2)
    h3 = mmid_call(p2[0], p2[1], h2, dinv_col, b2.reshape(1, H), W3)
    p3 = edge_call(src_p, dst_p, w_p, h3)

    sums, cnts = pl.pallas_call(
        _pool_body,
        grid=grid,
        in_specs=[_row_spec((BLK, H)), _row_spec((BLK, H)),
                  _row_spec((BLK, H)), _row_spec((BLK, 1)),
                  _const_spec((1, H)), _row_spec((BLK, 1))],
        out_specs=[_const_spec((G, H)), _const_spec((G, H))],
        out_shape=[jax.ShapeDtypeStruct((G, H), F32),
                   jax.ShapeDtypeStruct((G, H), F32)],
    )(p3[0], p3[1], h3, dinv_col, b3.reshape(1, H), batch_p)

    wr_pad = jnp.zeros((H, H), F32).at[:, :OUT].set(Wr)
    br_pad = jnp.zeros((1, H), F32).at[0, :OUT].set(br)
    out128 = pl.pallas_call(
        _head_body,
        out_shape=jax.ShapeDtypeStruct((G, H), F32),
    )(sums, cnts, wr_pad, br_pad)
    return out128[:, :OUT]


# speedup vs baseline: 17.7135x; 1.0074x over previous
import functools

import jax
import jax.numpy as jnp
from jax import lax
from jax.experimental import pallas as pl
from jax.experimental.pallas import tpu as pltpu
from jax.experimental.pallas import tpu_sc as plsc

NC = 2
NS = 16
NW = NC * NS
LANES = 16
BLK = 128
G = 16
F32 = jnp.float32


def _mesh():
    return plsc.VectorSubcoreMesh(core_axis_name="c", subcore_axis_name="s")


def _deg_body(dst_h, w_h, out_h, dst_v, w_v, zbuf, acc_deg):
    c = lax.axis_index("c")
    s = lax.axis_index("s")
    wid = c * NS + s
    np_ = acc_deg.shape[0]
    rpt = np_ // NS
    pltpu.sync_copy(dst_h.at[wid], dst_v)
    pltpu.sync_copy(w_h.at[wid], w_v)

    z = jnp.zeros((LANES,), F32)

    def zero_body(i, _):
        zbuf[pl.ds(i * LANES, LANES)] = z
        return 0

    lax.fori_loop(0, rpt // LANES, zero_body, 0)
    pltpu.sync_copy(zbuf, acc_deg.at[pl.ds(s * rpt, rpt)])
    plsc.subcore_barrier()

    nblk = dst_v.shape[0]

    def blk_body(j, _):
        pltpu.sync_copy(w_v.at[j], acc_deg.at[dst_v.at[j]], add=True)
        return 0

    lax.fori_loop(0, nblk, blk_body, 0)
    plsc.subcore_barrier()
    pltpu.sync_copy(acc_deg.at[pl.ds(s * rpt, rpt)],
                    out_h.at[c, pl.ds(s * rpt, rpt)])


CHUNK = 8
CPB = 2 * CHUNK


def _edge_body(src_h, dst_h, w_h, tab_h, out_h,
               rows0, rows1, src_c0, src_c1, dst_c0, dst_c1, w_c0, w_c1,
               acc, g0, g1, h0, h1, s0, s1, i0, i1):
    c = lax.axis_index("c")
    s = lax.axis_index("s")
    wid = c * NS + s
    np_ = acc.shape[0]
    rpt = np_ // NS
    cb = src_h.shape[1]
    ncp = cb // CPB

    rows = (rows0, rows1)
    srcc = (src_c0, src_c1)
    dstc = (dst_c0, dst_c1)
    wc = (w_c0, w_c1)
    gsem = (g0, g1)
    hsem = (h0, h1)
    ssem = (s0, s1)
    isem = (i0, i1)

    z = jnp.zeros((LANES,), F32)

    def zero_rows(i, _):
        for k in range(BLK // LANES):
            rows0[i, pl.ds(k * LANES, LANES)] = z
        return 0

    lax.fori_loop(0, BLK, zero_rows, 0)
    for t in range(rpt // BLK):
        pltpu.sync_copy(rows0, acc.at[pl.ds(s * rpt + t * BLK, BLK)])
    plsc.subcore_barrier()

    def mult(buf, wref, k):
        def grp(g, _):
            w16 = wref[k, pl.ds(g * LANES, LANES)]
            for l in range(LANES):
                wsc = w16[l]
                e = g * LANES + l
                for q in range(BLK // LANES):
                    sl = pl.ds(q * LANES, LANES)
                    buf[e, sl] = buf[e, sl] * wsc
            return 0

        lax.fori_loop(0, BLK // LANES, grp, 0)

    def pf(cidx, st):
        pltpu.async_copy(src_h.at[wid, pl.ds(cidx * CHUNK, CHUNK)],
                         srcc[st], isem[st])
        pltpu.async_copy(dst_h.at[wid, pl.ds(cidx * CHUNK, CHUNK)],
                         dstc[st], isem[st])
        pltpu.async_copy(w_h.at[wid, pl.ds(cidx * CHUNK, CHUNK)],
                         wc[st], isem[st])

    def pf_wait(st):
        pltpu.make_async_copy(src_h.at[wid, pl.ds(0, CHUNK)],
                              srcc[st], isem[st]).wait()
        pltpu.make_async_copy(dst_h.at[wid, pl.ds(0, CHUNK)],
                              dstc[st], isem[st]).wait()
        pltpu.make_async_copy(w_h.at[wid, pl.ds(0, CHUNK)],
                              wc[st], isem[st]).wait()

    pltpu.sync_copy(src_h.at[wid, pl.ds(0, CHUNK)], src_c0)
    pltpu.sync_copy(dst_h.at[wid, pl.ds(0, CHUNK)], dst_c0)
    pltpu.sync_copy(w_h.at[wid, pl.ds(0, CHUNK)], w_c0)
    HB = BLK // 2

    def g_issue(idxrow, buf, bi):
        pltpu.async_copy(tab_h.at[idxrow.at[pl.ds(0, HB)]],
                         buf.at[pl.ds(0, HB)], gsem[bi])
        pltpu.async_copy(tab_h.at[idxrow.at[pl.ds(HB, HB)]],
                         buf.at[pl.ds(HB, HB)], hsem[bi])

    def g_wait(idxrow, buf, bi):
        pltpu.make_async_copy(tab_h.at[idxrow.at[pl.ds(0, HB)]],
                              buf.at[pl.ds(0, HB)], gsem[bi]).wait()
        pltpu.make_async_copy(tab_h.at[idxrow.at[pl.ds(HB, HB)]],
                              buf.at[pl.ds(HB, HB)], hsem[bi]).wait()

    g_issue(src_c0.at[0], rows0, 0)

    def cpair(cp, _):
        for jj in range(CPB):
            st = jj // CHUNK
            k = jj % CHUNK
            b = jj % 2
            g_wait(srcc[st].at[k], rows[b], b)
            def drain():
                pltpu.make_async_copy(rows[1 - b], acc.at[dstc[st].at[k]],
                                      ssem[1 - b]).wait()
            if jj == 0:
                @pl.when(cp >= 1)
                def _():
                    drain()
            else:
                drain()
            if jj == 2:
                pf(2 * cp + 1, 1)
            if jj == 10:
                @pl.when(cp < ncp - 1)
                def _():
                    pf(2 * cp + 2, 0)
            if jj < CPB - 1:
                nst = (jj + 1) // CHUNK
                nk = (jj + 1) % CHUNK
                if jj == CHUNK - 1:
                    pf_wait(1)
                g_issue(srcc[nst].at[nk], rows[1 - b], 1 - b)
            else:
                @pl.when(cp < ncp - 1)
                def _():
                    pf_wait(0)
                    g_issue(src_c0.at[0], rows[1 - b], 1 - b)
            mult(rows[b], wc[st], k)
            pltpu.async_copy(rows[b], acc.at[dstc[st].at[k]], ssem[b],
                             add=True)
        return 0

    lax.fori_loop(0, ncp, cpair, 0)
    pltpu.make_async_copy(rows1, acc.at[dst_c1.at[CHUNK - 1]], s1).wait()
    plsc.subcore_barrier()
    pltpu.sync_copy(acc.at[pl.ds(s * rpt, rpt)],
                    out_h.at[c, pl.ds(s * rpt, rpt)])


def _m1_body(x_ref, degp_ref, w_ref, out_ref, dinv_ref):
    deg = 1.0 + jnp.sum(degp_ref[...], axis=0, keepdims=True)
    dinv_row = jnp.where(deg > 0, lax.rsqrt(deg), 0.0)
    n = dinv_row.shape[1]
    eye = (lax.broadcasted_iota(jnp.int32, (n, n), 0)
           == lax.broadcasted_iota(jnp.int32, (n, n), 1)).astype(F32)
    dinv_col = lax.dot_general(eye, dinv_row, (((1,), (1,)), ((), ())),
                               precision=lax.Precision.HIGHEST)
    dinv_ref[...] = dinv_col
    h = jnp.dot(x_ref[...], w_ref[...], preferred_element_type=F32,
                precision=lax.Precision.DEFAULT)
    out_ref[...] = dinv_col * h


def _mmid_body(pa_ref, pb_ref, hp_ref, dinv_ref, bias_ref, w_ref, out_ref):
    dinv = dinv_ref[...]
    o = dinv * (pa_ref[...] + pb_ref[...] + hp_ref[...]) + bias_ref[...]
    a = jnp.maximum(o, 0.0)
    h = jnp.dot(a, w_ref[...], preferred_element_type=F32,
                precision=lax.Precision.DEFAULT)
    out_ref[...] = dinv * h


def _pool_body(pa_ref, pb_ref, hp_ref, dinv_ref, bias_ref, batch_ref,
               wr_ref, br_ref, out_ref, sums_ref, cnts_ref):
    i = pl.program_id(0)
    nb = pl.num_programs(0)
    o = dinv_ref[...] * (pa_ref[...] + pb_ref[...] + hp_ref[...]) + bias_ref[...]
    bt = batch_ref[...]
    gids = lax.broadcasted_iota(jnp.int32, (bt.shape[0], G), 1)
    onehot = (bt == gids).astype(F32)
    dn = (((0,), (0,)), ((), ()))
    ps = lax.dot_general(onehot, o, dn, precision=lax.Precision.HIGHEST)
    pc = lax.dot_general(onehot, jnp.ones_like(o), dn,
                         precision=lax.Precision.HIGHEST)

    @pl.when(i == 0)
    def _():
        sums_ref[...] = jnp.zeros_like(sums_ref)
        cnts_ref[...] = jnp.zeros_like(cnts_ref)

    sums_ref[...] += ps
    cnts_ref[...] += pc

    @pl.when(i == nb - 1)
    def _():
        pooled = sums_ref[...] / jnp.maximum(cnts_ref[...], 1.0)
        oo = jnp.dot(pooled, wr_ref[...], preferred_element_type=F32,
                     precision=lax.Precision.DEFAULT) + br_ref[...]
        out_ref[...] = jnp.tanh(oo)


def _row_spec(blk):
    return pl.BlockSpec(blk, lambda i: (i, 0))


def _const_spec(blk):
    return pl.BlockSpec(blk, lambda i: (0, 0))


def kernel(x, edge_index, edge_weight, batch, W1, b1, W2, b2, W3, b3, Wr, br):
    N, D = x.shape
    E = edge_index.shape[1]
    H = W1.shape[1]
    OUT = Wr.shape[1]

    NP = -(-N // (NS * BLK)) * (NS * BLK)
    CB = CPB * -(-E // (NW * BLK * CPB))
    EP = NW * CB * BLK

    x_p = jnp.zeros((NP, D), F32).at[:N].set(x)
    src = edge_index[0]
    dst = edge_index[1]
    pad_e = EP - E
    fill = jnp.arange(pad_e, dtype=jnp.int32) % N
    src_p = jnp.concatenate([src, fill]).reshape(NW, CB, BLK)
    dst_p = jnp.concatenate([dst, fill]).reshape(NW, CB, BLK)
    w_p = jnp.concatenate(
        [edge_weight, jnp.zeros((pad_e,), F32)]).reshape(NW, CB, BLK)
    batch_p = jnp.concatenate(
        [batch.astype(jnp.int32), jnp.full((NP - N,), G, jnp.int32)]
    ).reshape(NP, 1)

    deg_call = pl.kernel(
        _deg_body,
        out_type=jax.ShapeDtypeStruct((NC, NP), F32),
        mesh=_mesh(),
        scratch_types=[
            pltpu.VMEM((CB, BLK), jnp.int32),
            pltpu.VMEM((CB, BLK), F32),
            pltpu.VMEM((NP // NS,), F32),
            pltpu.VMEM_SHARED((NP,), F32),
        ],
    )
    degp = deg_call(dst_p, w_p)

    edge_call = pl.kernel(
        _edge_body,
        out_type=jax.ShapeDtypeStruct((NC, NP, H), F32),
        mesh=_mesh(),
        scratch_types=[
            pltpu.VMEM((BLK, H), F32),
            pltpu.VMEM((BLK, H), F32),
            pltpu.VMEM((CHUNK, BLK), jnp.int32),
            pltpu.VMEM((CHUNK, BLK), jnp.int32),
            pltpu.VMEM((CHUNK, BLK), jnp.int32),
            pltpu.VMEM((CHUNK, BLK), jnp.int32),
            pltpu.VMEM((CHUNK, BLK), F32),
            pltpu.VMEM((CHUNK, BLK), F32),
            pltpu.VMEM_SHARED((NP, H), F32),
            pltpu.SemaphoreType.DMA,
            pltpu.SemaphoreType.DMA,
            pltpu.SemaphoreType.DMA,
            pltpu.SemaphoreType.DMA,
            pltpu.SemaphoreType.DMA,
            pltpu.SemaphoreType.DMA,
            pltpu.SemaphoreType.DMA,
            pltpu.SemaphoreType.DMA,
        ],
    )

    grid = (NP // BLK,)
    m1_call = pl.pallas_call(
        _m1_body,
        grid=grid,
        in_specs=[_row_spec((BLK, D)),
                  pl.BlockSpec((NC, BLK), lambda i: (0, i)),
                  _const_spec((D, H))],
        out_specs=[_row_spec((BLK, H)), _row_spec((BLK, 1))],
        out_shape=[jax.ShapeDtypeStruct((NP, H), F32),
                   jax.ShapeDtypeStruct((NP, 1), F32)],
    )
    mmid_call = pl.pallas_call(
        _mmid_body,
        grid=grid,
        in_specs=[_row_spec((BLK, H)), _row_spec((BLK, H)),
                  _row_spec((BLK, H)), _row_spec((BLK, 1)),
                  _const_spec((1, H)), _const_spec((H, H))],
        out_specs=_row_spec((BLK, H)),
        out_shape=jax.ShapeDtypeStruct((NP, H), F32),
    )

    h1, dinv_col = m1_call(x_p, degp, W1)
    p1 = edge_call(src_p, dst_p, w_p, h1)
    h2 = mmid_call(p1[0], p1[1], h1, dinv_col, b1.reshape(1, H), W2)
    p2 = edge_call(src_p, dst_p, w_p, h2)
    h3 = mmid_call(p2[0], p2[1], h2, dinv_col, b2.reshape(1, H), W3)
    p3 = edge_call(src_p, dst_p, w_p, h3)

    wr_pad = jnp.zeros((H, H), F32).at[:, :OUT].set(Wr)
    br_pad = jnp.zeros((1, H), F32).at[0, :OUT].set(br)
    out128 = pl.pallas_call(
        _pool_body,
        grid=grid,
        in_specs=[_row_spec((BLK, H)), _row_spec((BLK, H)),
                  _row_spec((BLK, H)), _row_spec((BLK, 1)),
                  _const_spec((1, H)), _row_spec((BLK, 1)),
                  _const_spec((H, H)), _const_spec((1, H))],
        out_specs=_const_spec((G, H)),
        out_shape=jax.ShapeDtypeStruct((G, H), F32),
        scratch_shapes=[pltpu.VMEM((G, H), F32), pltpu.VMEM((G, H), F32)],
    )(p3[0], p3[1], h3, dinv_col, b3.reshape(1, H), batch_p,
      wr_pad, br_pad)
    return out128[:, :OUT]


# speedup vs baseline: 17.7901x; 1.0043x over previous
import functools

import jax
import jax.numpy as jnp
from jax import lax
from jax.experimental import pallas as pl
from jax.experimental.pallas import tpu as pltpu
from jax.experimental.pallas import tpu_sc as plsc

NC = 2
NS = 16
NW = NC * NS
LANES = 16
BLK = 128
G = 16
F32 = jnp.float32


def _mesh():
    return plsc.VectorSubcoreMesh(core_axis_name="c", subcore_axis_name="s")


def _deg_body(dst_h, w_h, out_h, dst_v, w_v, zbuf, acc_deg):
    c = lax.axis_index("c")
    s = lax.axis_index("s")
    wid = c * NS + s
    np_ = acc_deg.shape[0]
    rpt = np_ // NS
    pltpu.sync_copy(dst_h.at[wid], dst_v)
    pltpu.sync_copy(w_h.at[wid], w_v)

    z = jnp.zeros((LANES,), F32)

    def zero_body(i, _):
        zbuf[pl.ds(i * LANES, LANES)] = z
        return 0

    lax.fori_loop(0, rpt // LANES, zero_body, 0)
    pltpu.sync_copy(zbuf, acc_deg.at[pl.ds(s * rpt, rpt)])
    plsc.subcore_barrier()

    nblk = dst_v.shape[0]

    def blk_body(j, _):
        pltpu.sync_copy(w_v.at[j], acc_deg.at[dst_v.at[j]], add=True)
        return 0

    lax.fori_loop(0, nblk, blk_body, 0)
    plsc.subcore_barrier()
    pltpu.sync_copy(acc_deg.at[pl.ds(s * rpt, rpt)],
                    out_h.at[c, pl.ds(s * rpt, rpt)])


CHUNK = 8
CPB = 2 * CHUNK


def _edge_body(src_h, dst_h, w_h, tab_h, out_h,
               rows0, rows1, src_c0, src_c1, dst_c0, dst_c1, w_c0, w_c1,
               acc, g0, g1, h0, h1, s0, s1, i0, i1):
    c = lax.axis_index("c")
    s = lax.axis_index("s")
    wid = c * NS + s
    np_ = acc.shape[0]
    rpt = np_ // NS
    cb = src_h.shape[1]
    ncp = cb // CPB

    rows = (rows0, rows1)
    srcc = (src_c0, src_c1)
    dstc = (dst_c0, dst_c1)
    wc = (w_c0, w_c1)
    gsem = (g0, g1)
    hsem = (h0, h1)
    ssem = (s0, s1)
    isem = (i0, i1)

    z = jnp.zeros((LANES,), F32)

    def zero_rows(i, _):
        for k in range(BLK // LANES):
            rows0[i, pl.ds(k * LANES, LANES)] = z
        return 0

    lax.fori_loop(0, BLK, zero_rows, 0)
    for t in range(rpt // BLK):
        pltpu.async_copy(rows0, acc.at[pl.ds(s * rpt + t * BLK, BLK)], s0)
    for t in range(rpt // BLK):
        pltpu.make_async_copy(
            rows0, acc.at[pl.ds(s * rpt + t * BLK, BLK)], s0).wait()
    plsc.subcore_barrier()

    def mult(buf, wref, k):
        def grp(g, _):
            w16 = wref[k, pl.ds(g * LANES, LANES)]
            for l in range(LANES):
                wsc = w16[l]
                e = g * LANES + l
                for q in range(BLK // LANES):
                    sl = pl.ds(q * LANES, LANES)
                    buf[e, sl] = buf[e, sl] * wsc
            return 0

        lax.fori_loop(0, BLK // LANES, grp, 0)

    def pf(cidx, st):
        pltpu.async_copy(src_h.at[wid, pl.ds(cidx * CHUNK, CHUNK)],
                         srcc[st], isem[st])
        pltpu.async_copy(dst_h.at[wid, pl.ds(cidx * CHUNK, CHUNK)],
                         dstc[st], isem[st])
        pltpu.async_copy(w_h.at[wid, pl.ds(cidx * CHUNK, CHUNK)],
                         wc[st], isem[st])

    def pf_wait(st):
        pltpu.make_async_copy(src_h.at[wid, pl.ds(0, CHUNK)],
                              srcc[st], isem[st]).wait()
        pltpu.make_async_copy(dst_h.at[wid, pl.ds(0, CHUNK)],
                              dstc[st], isem[st]).wait()
        pltpu.make_async_copy(w_h.at[wid, pl.ds(0, CHUNK)],
                              wc[st], isem[st]).wait()

    pltpu.sync_copy(src_h.at[wid, pl.ds(0, CHUNK)], src_c0)
    pltpu.sync_copy(dst_h.at[wid, pl.ds(0, CHUNK)], dst_c0)
    pltpu.sync_copy(w_h.at[wid, pl.ds(0, CHUNK)], w_c0)
    HB = BLK // 2

    def g_issue(idxrow, buf, bi):
        pltpu.async_copy(tab_h.at[idxrow.at[pl.ds(0, HB)]],
                         buf.at[pl.ds(0, HB)], gsem[bi])
        pltpu.async_copy(tab_h.at[idxrow.at[pl.ds(HB, HB)]],
                         buf.at[pl.ds(HB, HB)], hsem[bi])

    def g_wait(idxrow, buf, bi):
        pltpu.make_async_copy(tab_h.at[idxrow.at[pl.ds(0, HB)]],
                              buf.at[pl.ds(0, HB)], gsem[bi]).wait()
        pltpu.make_async_copy(tab_h.at[idxrow.at[pl.ds(HB, HB)]],
                              buf.at[pl.ds(HB, HB)], hsem[bi]).wait()

    g_issue(src_c0.at[0], rows0, 0)

    def cpair(cp, _):
        for jj in range(CPB):
            st = jj // CHUNK
            k = jj % CHUNK
            b = jj % 2
            g_wait(srcc[st].at[k], rows[b], b)
            def drain():
                pltpu.make_async_copy(rows[1 - b], acc.at[dstc[st].at[k]],
                                      ssem[1 - b]).wait()
            if jj == 0:
                @pl.when(cp >= 1)
                def _():
                    drain()
            else:
                drain()
            if jj == 2:
                pf(2 * cp + 1, 1)
            if jj == 10:
                @pl.when(cp < ncp - 1)
                def _():
                    pf(2 * cp + 2, 0)
            if jj < CPB - 1:
                nst = (jj + 1) // CHUNK
                nk = (jj + 1) % CHUNK
                if jj == CHUNK - 1:
                    pf_wait(1)
                g_issue(srcc[nst].at[nk], rows[1 - b], 1 - b)
            else:
                @pl.when(cp < ncp - 1)
                def _():
                    pf_wait(0)
                    g_issue(src_c0.at[0], rows[1 - b], 1 - b)
            mult(rows[b], wc[st], k)
            pltpu.async_copy(rows[b], acc.at[dstc[st].at[k]], ssem[b],
                             add=True)
        return 0

    lax.fori_loop(0, ncp, cpair, 0)
    pltpu.make_async_copy(rows1, acc.at[dst_c1.at[CHUNK - 1]], s1).wait()
    plsc.subcore_barrier()
    pltpu.sync_copy(acc.at[pl.ds(s * rpt, rpt)],
                    out_h.at[c, pl.ds(s * rpt, rpt)])


def _m1_body(x_ref, degp_ref, w_ref, out_ref, dinv_ref):
    deg = 1.0 + jnp.sum(degp_ref[...], axis=0, keepdims=True)
    dinv_row = jnp.where(deg > 0, lax.rsqrt(deg), 0.0)
    n = dinv_row.shape[1]
    eye = (lax.broadcasted_iota(jnp.int32, (n, n), 0)
           == lax.broadcasted_iota(jnp.int32, (n, n), 1)).astype(F32)
    dinv_col = lax.dot_general(eye, dinv_row, (((1,), (1,)), ((), ())),
                               precision=lax.Precision.HIGHEST)
    dinv_ref[...] = dinv_col
    h = jnp.dot(x_ref[...], w_ref[...], preferred_element_type=F32,
                precision=lax.Precision.DEFAULT)
    out_ref[...] = dinv_col * h


def _mmid_body(pa_ref, pb_ref, hp_ref, dinv_ref, bias_ref, w_ref, out_ref):
    dinv = dinv_ref[...]
    o = dinv * (pa_ref[...] + pb_ref[...] + hp_ref[...]) + bias_ref[...]
    a = jnp.maximum(o, 0.0)
    h = jnp.dot(a, w_ref[...], preferred_element_type=F32,
                precision=lax.Precision.DEFAULT)
    out_ref[...] = dinv * h


def _pool_body(pa_ref, pb_ref, hp_ref, dinv_ref, bias_ref, batch_ref,
               wr_ref, br_ref, out_ref, sums_ref, cnts_ref):
    i = pl.program_id(0)
    nb = pl.num_programs(0)
    o = dinv_ref[...] * (pa_ref[...] + pb_ref[...] + hp_ref[...]) + bias_ref[...]
    bt = batch_ref[...]
    gids = lax.broadcasted_iota(jnp.int32, (bt.shape[0], G), 1)
    onehot = (bt == gids).astype(F32)
    dn = (((0,), (0,)), ((), ()))
    ps = lax.dot_general(onehot, o, dn, precision=lax.Precision.HIGHEST)
    pc = lax.dot_general(onehot, jnp.ones_like(o), dn,
                         precision=lax.Precision.HIGHEST)

    @pl.when(i == 0)
    def _():
        sums_ref[...] = jnp.zeros_like(sums_ref)
        cnts_ref[...] = jnp.zeros_like(cnts_ref)

    sums_ref[...] += ps
    cnts_ref[...] += pc

    @pl.when(i == nb - 1)
    def _():
        pooled = sums_ref[...] / jnp.maximum(cnts_ref[...], 1.0)
        oo = jnp.dot(pooled, wr_ref[...], preferred_element_type=F32,
                     precision=lax.Precision.DEFAULT) + br_ref[...]
        out_ref[...] = jnp.tanh(oo)


def _row_spec(blk):
    return pl.BlockSpec(blk, lambda i: (i, 0))


def _const_spec(blk):
    return pl.BlockSpec(blk, lambda i: (0, 0))


def kernel(x, edge_index, edge_weight, batch, W1, b1, W2, b2, W3, b3, Wr, br):
    N, D = x.shape
    E = edge_index.shape[1]
    H = W1.shape[1]
    OUT = Wr.shape[1]

    NP = -(-N // (NS * BLK)) * (NS * BLK)
    CB = CPB * -(-E // (NW * BLK * CPB))
    EP = NW * CB * BLK

    x_p = jnp.zeros((NP, D), F32).at[:N].set(x)
    src = edge_index[0]
    dst = edge_index[1]
    pad_e = EP - E
    fill = jnp.arange(pad_e, dtype=jnp.int32) % N
    src_p = jnp.concatenate([src, fill]).reshape(NW, CB, BLK)
    dst_p = jnp.concatenate([dst, fill]).reshape(NW, CB, BLK)
    w_p = jnp.concatenate(
        [edge_weight, jnp.zeros((pad_e,), F32)]).reshape(NW, CB, BLK)
    batch_p = jnp.concatenate(
        [batch.astype(jnp.int32), jnp.full((NP - N,), G, jnp.int32)]
    ).reshape(NP, 1)

    deg_call = pl.kernel(
        _deg_body,
        out_type=jax.ShapeDtypeStruct((NC, NP), F32),
        mesh=_mesh(),
        scratch_types=[
            pltpu.VMEM((CB, BLK), jnp.int32),
            pltpu.VMEM((CB, BLK), F32),
            pltpu.VMEM((NP // NS,), F32),
            pltpu.VMEM_SHARED((NP,), F32),
        ],
    )
    degp = deg_call(dst_p, w_p)

    edge_call = pl.kernel(
        _edge_body,
        out_type=jax.ShapeDtypeStruct((NC, NP, H), F32),
        mesh=_mesh(),
        scratch_types=[
            pltpu.VMEM((BLK, H), F32),
            pltpu.VMEM((BLK, H), F32),
            pltpu.VMEM((CHUNK, BLK), jnp.int32),
            pltpu.VMEM((CHUNK, BLK), jnp.int32),
            pltpu.VMEM((CHUNK, BLK), jnp.int32),
            pltpu.VMEM((CHUNK, BLK), jnp.int32),
            pltpu.VMEM((CHUNK, BLK), F32),
            pltpu.VMEM((CHUNK, BLK), F32),
            pltpu.VMEM_SHARED((NP, H), F32),
            pltpu.SemaphoreType.DMA,
            pltpu.SemaphoreType.DMA,
            pltpu.SemaphoreType.DMA,
            pltpu.SemaphoreType.DMA,
            pltpu.SemaphoreType.DMA,
            pltpu.SemaphoreType.DMA,
            pltpu.SemaphoreType.DMA,
            pltpu.SemaphoreType.DMA,
        ],
    )

    grid = (NP // BLK,)
    m1_call = pl.pallas_call(
        _m1_body,
        grid=grid,
        in_specs=[_row_spec((BLK, D)),
                  pl.BlockSpec((NC, BLK), lambda i: (0, i)),
                  _const_spec((D, H))],
        out_specs=[_row_spec((BLK, H)), _row_spec((BLK, 1))],
        out_shape=[jax.ShapeDtypeStruct((NP, H), F32),
                   jax.ShapeDtypeStruct((NP, 1), F32)],
    )
    mmid_call = pl.pallas_call(
        _mmid_body,
        grid=grid,
        in_specs=[_row_spec((BLK, H)), _row_spec((BLK, H)),
                  _row_spec((BLK, H)), _row_spec((BLK, 1)),
                  _const_spec((1, H)), _const_spec((H, H))],
        out_specs=_row_spec((BLK, H)),
        out_shape=jax.ShapeDtypeStruct((NP, H), F32),
    )

    h1, dinv_col = m1_call(x_p, degp, W1)
    p1 = edge_call(src_p, dst_p, w_p, h1)
    h2 = mmid_call(p1[0], p1[1], h1, dinv_col, b1.reshape(1, H), W2)
    p2 = edge_call(src_p, dst_p, w_p, h2)
    h3 = mmid_call(p2[0], p2[1], h2, dinv_col, b2.reshape(1, H), W3)
    p3 = edge_call(src_p, dst_p, w_p, h3)

    wr_pad = jnp.zeros((H, H), F32).at[:, :OUT].set(Wr)
    br_pad = jnp.zeros((1, H), F32).at[0, :OUT].set(br)
    out128 = pl.pallas_call(
        _pool_body,
        grid=grid,
        in_specs=[_row_spec((BLK, H)), _row_spec((BLK, H)),
                  _row_spec((BLK, H)), _row_spec((BLK, 1)),
                  _const_spec((1, H)), _row_spec((BLK, 1)),
                  _const_spec((H, H)), _const_spec((1, H))],
        out_specs=_const_spec((G, H)),
        out_shape=jax.ShapeDtypeStruct((G, H), F32),
        scratch_shapes=[pltpu.VMEM((G, H), F32), pltpu.VMEM((G, H), F32)],
    )(p3[0], p3[1], h3, dinv_col, b3.reshape(1, H), batch_p,
      wr_pad, br_pad)
    return out128[:, :OUT]
